# baseline (speedup 1.0000x reference)
import functools

import jax
import jax.numpy as jnp
from jax import lax
from jax.experimental import pallas as pl
from jax.experimental.pallas import tpu as pltpu
from jax.experimental.pallas import tpu_sc as plsc

_N = 10000
_E = 160000
_LAT = 128
_H1 = 256
_H2 = 128

def _when(pred, fn):
    if isinstance(pred, bool):
        if pred:
            fn()
    else:
        pl.when(pred)(fn)


_NC, _NS = 2, 16
_NW = _NC * _NS

_NB = 10
_NBLK = _N // _NB
_EBLK = 2000
_EB = _E // _EBLK

_CG = 200
_CS = 40
_ESLAB = _E // 2


def _pack_bf16_pair(lo_f, hi_f):
    lb = lax.bitcast_convert_type(lo_f, jnp.int32)
    hb = lax.bitcast_convert_type(hi_f, jnp.int32)
    lr = (lb + 0x7FFF + ((lb >> 16) & 1)) >> 16
    hr = (hb + 0x7FFF + ((hb >> 16) & 1)) >> 16
    return (lr & jnp.int32(0xFFFF)) | (hr << 16)


def _unpack_bf16_pair(w):
    lo = lax.bitcast_convert_type(w << 16, jnp.float32)
    hi = lax.bitcast_convert_type(w & jnp.int32(-65536), jnp.float32)
    return lo, hi


def _prep_body(x_ref, wen_ref, ben_ref, wbl_ref, wbh_ref, wcl_ref, wch_ref,
               nodes_ref, s_ref, r_ref):
    nb = jnp.dot(x_ref[...], wen_ref[...], preferred_element_type=jnp.float32)
    nb = nb + ben_ref[...]
    nodes_ref[...] = nb
    s_ref[...] = _pack_bf16_pair(
        jnp.dot(nb, wbl_ref[...], preferred_element_type=jnp.float32),
        jnp.dot(nb, wbh_ref[...], preferred_element_type=jnp.float32))
    r_ref[...] = _pack_bf16_pair(
        jnp.dot(nb, wcl_ref[...], preferred_element_type=jnp.float32),
        jnp.dot(nb, wch_ref[...], preferred_element_type=jnp.float32))


def _prep(x, W_en, b_en, We1b, We1c):
    full = lambda shape: pl.BlockSpec(shape, lambda i: (0, 0))
    return pl.pallas_call(
        _prep_body,
        grid=(_NB,),
        in_specs=[
            pl.BlockSpec((_NBLK, _LAT), lambda i: (i, 0)),
            full((_LAT, _LAT)),
            full((1, _LAT)),
            full((_LAT, _H2)),
            full((_LAT, _H2)),
            full((_LAT, _H2)),
            full((_LAT, _H2)),
        ],
        out_specs=[
            pl.BlockSpec((_NBLK, _LAT), lambda i: (i, 0)),
            pl.BlockSpec((_NBLK, _H2), lambda i: (i, 0)),
            pl.BlockSpec((_NBLK, _H2), lambda i: (i, 0)),
        ],
        out_shape=[
            jax.ShapeDtypeStruct((_N, _LAT), jnp.float32),
            jax.ShapeDtypeStruct((_N, _H2), jnp.int32),
            jax.ShapeDtypeStruct((_N, _H2), jnp.int32),
        ],
    )(x, W_en, b_en.reshape(1, _LAT), We1b[:, :_H2], We1b[:, _H2:],
      We1c[:, :_H2], We1c[:, _H2:])


def _gather_body(s_hbm, r_hbm, snd_hbm, rcv_hbm, gs_hbm, gr_hbm,
                 idxs0_v, idxs1_v, idxr0_v, idxr1_v,
                 bufs0_v, bufs1_v, bufr0_v, bufr1_v,
                 sg0, sg1, sw0, sw1):
    wid = lax.axis_index("s") * _NC + lax.axis_index("c")
    per_w = _E // _NW
    nchunks = per_w // _CG
    base = wid * per_w
    sg = (sg0, sg1)
    sw = (sw0, sw1)
    idxs = (idxs0_v, idxs1_v)
    idxr = (idxr0_v, idxr1_v)
    bufs = (bufs0_v, bufs1_v)
    bufr = (bufr0_v, bufr1_v)

    def launch(slot, off, drain_w):
        def _drain():
            pltpu.make_async_copy(bufs[slot], gs_hbm.at[pl.ds(base, _CG)],
                                  sw[slot]).wait()
            pltpu.make_async_copy(bufr[slot], gr_hbm.at[pl.ds(base, _CG)],
                                  sw[slot]).wait()

        _when(drain_w, _drain)
        pltpu.sync_copy(snd_hbm.at[pl.ds(off, _CG)], idxs[slot])
        pltpu.sync_copy(rcv_hbm.at[pl.ds(off, _CG)], idxr[slot])
        pltpu.async_copy(s_hbm.at[idxs[slot]], bufs[slot], sg[slot])
        pltpu.async_copy(r_hbm.at[idxr[slot]], bufr[slot], sg[slot])

    def finish(slot, off):
        pltpu.make_async_copy(s_hbm.at[idxs[slot]], bufs[slot], sg[slot]).wait()
        pltpu.make_async_copy(r_hbm.at[idxr[slot]], bufr[slot], sg[slot]).wait()
        pltpu.async_copy(bufs[slot], gs_hbm.at[pl.ds(off, _CG)], sw[slot])
        pltpu.async_copy(bufr[slot], gr_hbm.at[pl.ds(off, _CG)], sw[slot])

    launch(0, base, False)

    def pair(i, carry):
        off0 = base + (2 * i) * _CG
        launch(1, off0 + _CG, i > 0)
        finish(0, off0)
        launch(0, off0 + 2 * _CG, True)
        finish(1, off0 + _CG)
        return carry

    lax.fori_loop(0, (nchunks - 1) // 2, pair, 0)
    finish(0, base + (nchunks - 1) * _CG)
    for slot in (0, 1):
        pltpu.make_async_copy(bufs[slot], gs_hbm.at[pl.ds(base, _CG)],
                              sw[slot]).wait()
        pltpu.make_async_copy(bufr[slot], gr_hbm.at[pl.ds(base, _CG)],
                              sw[slot]).wait()


def _gather(S, R, senders, receivers):
    mesh = plsc.VectorSubcoreMesh(core_axis_name="c", subcore_axis_name="s",
                                  num_cores=_NC, num_subcores=_NS)
    kfn = pl.kernel(
        _gather_body,
        out_type=[jax.ShapeDtypeStruct((_E, _H2), jnp.int32),
                  jax.ShapeDtypeStruct((_E, _H2), jnp.int32)],
        mesh=mesh,
        scratch_types=[
            pltpu.VMEM((_CG,), jnp.int32),
            pltpu.VMEM((_CG,), jnp.int32),
            pltpu.VMEM((_CG,), jnp.int32),
            pltpu.VMEM((_CG,), jnp.int32),
            pltpu.VMEM((_CG, _H2), jnp.int32),
            pltpu.VMEM((_CG, _H2), jnp.int32),
            pltpu.VMEM((_CG, _H2), jnp.int32),
            pltpu.VMEM((_CG, _H2), jnp.int32),
            pltpu.SemaphoreType.DMA,
            pltpu.SemaphoreType.DMA,
            pltpu.SemaphoreType.DMA,
            pltpu.SemaphoreType.DMA,
        ],
    )
    return kfn(S, R, senders, receivers)


def _edge_body(gs_ref, gr_ref, ea_ref, wee_ref, bee_ref,
               we1al_ref, we1ah_ref, be1_ref,
               we2l_ref, we2h_ref, be2_ref, out_ref):
    bee = bee_ref[...]
    ea = ea_ref[...]
    slo, shi = _unpack_bf16_pair(gs_ref[...])
    rlo, rhi = _unpack_bf16_pair(gr_ref[...])

    weal = jnp.dot(wee_ref[...], we1al_ref[...], preferred_element_type=jnp.float32)
    weah = jnp.dot(wee_ref[...], we1ah_ref[...], preferred_element_type=jnp.float32)
    bl = jnp.dot(bee, we1al_ref[...], preferred_element_type=jnp.float32)
    bh = jnp.dot(bee, we1ah_ref[...], preferred_element_type=jnp.float32)
    bl = bl + be1_ref[:, :_H2]
    bh = bh + be1_ref[:, _H2:]

    hlo = slo + rlo + jnp.dot(ea, weal, preferred_element_type=jnp.float32) + bl
    hhi = shi + rhi + jnp.dot(ea, weah, preferred_element_type=jnp.float32) + bh
    hlo = jnp.maximum(hlo, 0.0).astype(jnp.bfloat16)
    hhi = jnp.maximum(hhi, 0.0).astype(jnp.bfloat16)
    out = jnp.dot(hlo, we2l_ref[...].astype(jnp.bfloat16),
                  preferred_element_type=jnp.float32)
    out = out + jnp.dot(hhi, we2h_ref[...].astype(jnp.bfloat16),
                        preferred_element_type=jnp.float32)
    out_ref[...] = out + be2_ref[...]


def _edge_mlp(slab, Gs, Gr, edge_attr, W_ee, b_ee, We1a, be1, We2, be2):
    full = lambda shape: pl.BlockSpec(shape, lambda i: (0, 0))
    nblk = _ESLAB // _EBLK
    off = slab * nblk
    return pl.pallas_call(
        _edge_body,
        grid=(nblk,),
        in_specs=[
            pl.BlockSpec((_EBLK, _H2), lambda i: (i + off, 0)),
            pl.BlockSpec((_EBLK, _H2), lambda i: (i + off, 0)),
            pl.BlockSpec((_EBLK, 16), lambda i: (i + off, 0)),
            full((16, _LAT)),
            full((1, _LAT)),
            full((_LAT, _H2)),
            full((_LAT, _H2)),
            full((1, _H1)),
            full((_H2, _H2)),
            full((_H2, _H2)),
            full((1, _H2)),
        ],
        out_specs=pl.BlockSpec((_EBLK, _H2), lambda i: (i, 0)),
        out_shape=jax.ShapeDtypeStruct((_ESLAB, _H2), jnp.float32),
    )(Gs, Gr, edge_attr, W_ee, b_ee.reshape(1, _LAT),
      We1a[:, :_H2], We1a[:, _H2:], be1.reshape(1, _H1),
      We2[:_H2], We2[_H2:], be2.reshape(1, _H2))


_NPT = 632
_NPAD = _NS * _NPT


def _scatter_body(slab_off, en_hbm, idx2_hbm, out_hbm, acc_sh,
                  ebuf0, ebuf1, idx0_v, idx1_v,
                  sl0, sl1, si0, si1, ss0, ss1):
    c = lax.axis_index("c")
    s = lax.axis_index("s")
    ebuf = (ebuf0, ebuf1)
    idxm = (idx0_v, idx1_v)
    slm = (sl0, sl1)
    sim = (si0, si1)
    ssm = (ss0, ss1)

    def zrow(rr, carry):
        for l in range(_H2 // 16):
            ebuf0[rr, pl.ds(l * 16, 16)] = jnp.zeros((16,), jnp.float32)
        return carry

    lax.fori_loop(0, _CS, zrow, 0)
    for k in range(_NPT // _CS):
        pltpu.sync_copy(ebuf0, acc_sh.at[pl.ds(s * _NPT + k * _CS, _CS)])
    rem = _NPT % _CS
    if rem:
        pltpu.sync_copy(ebuf0.at[pl.ds(0, rem)],
                        acc_sh.at[pl.ds(s * _NPT + (_NPT // _CS) * _CS, rem)])
    plsc.subcore_barrier()

    per_tile = _ESLAB // _NS
    nchunks = per_tile // _CS
    base = s * per_tile

    def launch(slot, off, drain):
        _when(drain, lambda: pltpu.make_async_copy(
            ebuf[slot], acc_sh.at[idxm[slot]], ssm[slot]).wait())
        pltpu.async_copy(idx2_hbm.at[pl.ds(c * _E + slab_off + off, _CS)],
                         idxm[slot], sim[slot])
        pltpu.async_copy(en_hbm.at[pl.ds(off, _CS)], ebuf[slot], slm[slot])

    def finish(slot):
        pltpu.make_async_copy(idx2_hbm.at[pl.ds(c * _E + base, _CS)],
                              idxm[slot], sim[slot]).wait()
        pltpu.make_async_copy(en_hbm.at[pl.ds(base, _CS)], ebuf[slot],
                              slm[slot]).wait()
        pltpu.async_copy(ebuf[slot], acc_sh.at[idxm[slot]],
                         ssm[slot], add=True)

    launch(0, base, False)

    def pair(i, carry):
        off0 = base + (2 * i) * _CS
        launch(1, off0 + _CS, i > 0)
        finish(0)
        launch(0, off0 + 2 * _CS, True)
        finish(1)
        return carry

    lax.fori_loop(0, (nchunks - 1) // 2, pair, 0)
    finish(0)
    pltpu.make_async_copy(ebuf0, acc_sh.at[idx0_v], ssm[0]).wait()
    pltpu.make_async_copy(ebuf1, acc_sh.at[idx1_v], ssm[1]).wait()
    plsc.subcore_barrier()
    pltpu.sync_copy(acc_sh.at[pl.ds(s * _NPT, _NPT)],
                    out_hbm.at[c, pl.ds(s * _NPT, _NPT)])


def _scatter(slab, edges_new, idx2):
    mesh = plsc.VectorSubcoreMesh(core_axis_name="c", subcore_axis_name="s",
                                  num_cores=_NC, num_subcores=_NS)
    kfn = pl.kernel(
        functools.partial(_scatter_body, slab * _ESLAB),
        out_type=jax.ShapeDtypeStruct((2, _NPAD, _H2), jnp.float32),
        mesh=mesh,
        scratch_types=[
            pltpu.VMEM_SHARED((_NPAD, _H2), jnp.float32),
            pltpu.VMEM((_CS, _H2), jnp.float32),
            pltpu.VMEM((_CS, _H2), jnp.float32),
            pltpu.VMEM((_CS,), jnp.int32),
            pltpu.VMEM((_CS,), jnp.int32),
            pltpu.SemaphoreType.DMA,
            pltpu.SemaphoreType.DMA,
            pltpu.SemaphoreType.DMA,
            pltpu.SemaphoreType.DMA,
            pltpu.SemaphoreType.DMA,
            pltpu.SemaphoreType.DMA,
        ],
    )
    return kfn(edges_new, idx2)


def _node_body(nodes_ref, saA_ref, raA_ref, saB_ref, raB_ref,
               wn1a_ref, wn1b_ref, wn1c_ref, bn1_ref,
               wn2_ref, bn2_ref, wg1a_ref, wg1b_ref, bg1_ref, wg2_ref, bg2_ref,
               wg3_ref, bg3_ref, out_ref, s1_acc, ea_acc):
    i = pl.program_id(0)

    @pl.when(i == 0)
    def _init():
        s1_acc[...] = jnp.zeros_like(s1_acc)
        ea_acc[...] = jnp.zeros_like(ea_acc)

    sa = saA_ref[0] + saB_ref[0]
    ra = raA_ref[0] + raB_ref[0]
    h = jnp.dot(nodes_ref[...].astype(jnp.bfloat16),
                wn1a_ref[...].astype(jnp.bfloat16),
                preferred_element_type=jnp.float32)
    h = h + jnp.dot(sa.astype(jnp.bfloat16),
                    wn1b_ref[...].astype(jnp.bfloat16),
                    preferred_element_type=jnp.float32)
    h = h + jnp.dot(ra.astype(jnp.bfloat16),
                    wn1c_ref[...].astype(jnp.bfloat16),
                    preferred_element_type=jnp.float32)
    h = jnp.maximum(h + bn1_ref[...], 0.0)
    s1_acc[...] = s1_acc[...] + jnp.sum(h, axis=0, keepdims=True)
    ea_acc[...] = ea_acc[...] + jnp.sum(sa, axis=0, keepdims=True)

    @pl.when(i == _NB - 1)
    def _final():
        node_agg = jnp.dot(s1_acc[...], wn2_ref[...],
                           preferred_element_type=jnp.float32)
        node_agg = node_agg + jnp.float32(_N) * bn2_ref[...]
        edge_agg = ea_acc[...]
        hg = jnp.dot(node_agg, wg1a_ref[...], preferred_element_type=jnp.float32)
        hg = hg + jnp.dot(edge_agg, wg1b_ref[...], preferred_element_type=jnp.float32)
        hg = jnp.maximum(hg + bg1_ref[...], 0.0)
        hg2 = jnp.dot(hg, wg2_ref[...], preferred_element_type=jnp.float32)
        hg2 = jnp.maximum(hg2 + bg2_ref[...], 0.0)
        out_ref[...] = (jnp.dot(hg2, wg3_ref[...], preferred_element_type=jnp.float32)
                        + bg3_ref[...])


def _node_global(nodes, aggsA, aggsB, Wn1a, Wn1b, Wn1c, bn1, Wn2, bn2,
                 Wg1a, Wg1b, bg1, Wg2, bg2, Wg3, bg3):
    full = lambda shape: pl.BlockSpec(shape, lambda i: (0, 0))
    return pl.pallas_call(
        _node_body,
        grid=(_NB,),
        in_specs=[
            pl.BlockSpec((_NBLK, _LAT), lambda i: (i, 0)),
            pl.BlockSpec((1, _NBLK, _H2), lambda i: (0, i, 0)),
            pl.BlockSpec((1, _NBLK, _H2), lambda i: (1, i, 0)),
            pl.BlockSpec((1, _NBLK, _H2), lambda i: (0, i, 0)),
            pl.BlockSpec((1, _NBLK, _H2), lambda i: (1, i, 0)),
            full((_LAT, _H1)),
            full((_H2, _H1)),
            full((_H2, _H1)),
            full((1, _H1)),
            full((_H1, _H2)),
            full((1, _H2)),
            full((_H2, _H1)),
            full((_H2, _H1)),
            full((1, _H1)),
            full((_H1, _H2)),
            full((1, _H2)),
            full((_H2, 1)),
            full((1, 1)),
        ],
        out_specs=pl.BlockSpec((1, 1), lambda i: (0, 0)),
        out_shape=jax.ShapeDtypeStruct((1, 1), jnp.float32),
        scratch_shapes=[
            pltpu.VMEM((1, _H1), jnp.float32),
            pltpu.VMEM((1, _H2), jnp.float32),
        ],
    )(nodes, aggsA, aggsA, aggsB, aggsB, Wn1a, Wn1b, Wn1c, bn1.reshape(1, _H1),
      Wn2, bn2.reshape(1, _H2), Wg1a, Wg1b, bg1.reshape(1, _H1),
      Wg2, bg2.reshape(1, _H2), Wg3, bg3.reshape(1, 1))


def kernel(x, edge_attr, senders, receivers, W_en, b_en, W_ee, b_ee,
           We1, be1, We2, be2, Wn1, bn1, Wn2, bn2,
           Wg1, bg1, Wg2, bg2, Wg3, bg3):
    We1a = We1[:_LAT]
    We1b = We1[_LAT:2 * _LAT]
    We1c = We1[2 * _LAT:3 * _LAT]
    Wn1a = Wn1[:_LAT]
    Wn1b = Wn1[_LAT:2 * _LAT]
    Wn1c = Wn1[2 * _LAT:3 * _LAT]
    Wg1a = Wg1[:_H2]
    Wg1b = Wg1[_H2:2 * _H2]

    nodes, S32, R32 = _prep(x, W_en, b_en, We1b, We1c)
    Gs32, Gr32 = _gather(S32, R32, senders, receivers)
    idx2 = jnp.concatenate([senders, receivers])
    enA = _edge_mlp(0, Gs32, Gr32, edge_attr, W_ee, b_ee, We1a, be1, We2, be2)
    aggsA = _scatter(0, enA, idx2)
    enB = _edge_mlp(1, Gs32, Gr32, edge_attr, W_ee, b_ee, We1a, be1, We2, be2)
    aggsB = _scatter(1, enB, idx2)
    out = _node_global(nodes, aggsA, aggsB, Wn1a, Wn1b, Wn1c, bn1, Wn2, bn2,
                       Wg1a, Wg1b, bg1, Wg2, bg2, Wg3, bg3)
    return out


# speedup vs baseline: 1.0210x; 1.0210x over previous
import functools

import jax
import jax.numpy as jnp
from jax import lax
from jax.experimental import pallas as pl
from jax.experimental.pallas import tpu as pltpu
from jax.experimental.pallas import tpu_sc as plsc

_N = 10000
_E = 160000
_LAT = 128
_H1 = 256
_H2 = 128

def _when(pred, fn):
    if isinstance(pred, bool):
        if pred:
            fn()
    else:
        pl.when(pred)(fn)


_NC, _NS = 2, 16
_NW = _NC * _NS

_NB = 10
_NBLK = _N // _NB
_EBLK = 2000
_EB = _E // _EBLK

_CG = 200
_CS = 80
_ESLAB = _E // 2


def _pack_bf16_pair(lo_f, hi_f):
    lb = lax.bitcast_convert_type(lo_f, jnp.int32)
    hb = lax.bitcast_convert_type(hi_f, jnp.int32)
    lr = (lb + 0x7FFF + ((lb >> 16) & 1)) >> 16
    hr = (hb + 0x7FFF + ((hb >> 16) & 1)) >> 16
    return (lr & jnp.int32(0xFFFF)) | (hr << 16)


def _unpack_bf16_pair(w):
    lo = lax.bitcast_convert_type(w << 16, jnp.float32)
    hi = lax.bitcast_convert_type(w & jnp.int32(-65536), jnp.float32)
    return lo, hi


def _prep_body(x_ref, wen_ref, ben_ref, wbl_ref, wbh_ref, wcl_ref, wch_ref,
               nodes_ref, s_ref, r_ref):
    nb = jnp.dot(x_ref[...], wen_ref[...], preferred_element_type=jnp.float32)
    nb = nb + ben_ref[...]
    nodes_ref[...] = nb
    s_ref[...] = _pack_bf16_pair(
        jnp.dot(nb, wbl_ref[...], preferred_element_type=jnp.float32),
        jnp.dot(nb, wbh_ref[...], preferred_element_type=jnp.float32))
    r_ref[...] = _pack_bf16_pair(
        jnp.dot(nb, wcl_ref[...], preferred_element_type=jnp.float32),
        jnp.dot(nb, wch_ref[...], preferred_element_type=jnp.float32))


def _prep(x, W_en, b_en, We1b, We1c):
    full = lambda shape: pl.BlockSpec(shape, lambda i: (0, 0))
    return pl.pallas_call(
        _prep_body,
        grid=(_NB,),
        in_specs=[
            pl.BlockSpec((_NBLK, _LAT), lambda i: (i, 0)),
            full((_LAT, _LAT)),
            full((1, _LAT)),
            full((_LAT, _H2)),
            full((_LAT, _H2)),
            full((_LAT, _H2)),
            full((_LAT, _H2)),
        ],
        out_specs=[
            pl.BlockSpec((_NBLK, _LAT), lambda i: (i, 0)),
            pl.BlockSpec((_NBLK, _H2), lambda i: (i, 0)),
            pl.BlockSpec((_NBLK, _H2), lambda i: (i, 0)),
        ],
        out_shape=[
            jax.ShapeDtypeStruct((_N, _LAT), jnp.float32),
            jax.ShapeDtypeStruct((_N, _H2), jnp.int32),
            jax.ShapeDtypeStruct((_N, _H2), jnp.int32),
        ],
    )(x, W_en, b_en.reshape(1, _LAT), We1b[:, :_H2], We1b[:, _H2:],
      We1c[:, :_H2], We1c[:, _H2:])


def _gather_body(s_hbm, r_hbm, snd_hbm, rcv_hbm, gs_hbm, gr_hbm,
                 idxs0_v, idxs1_v, idxr0_v, idxr1_v,
                 bufs0_v, bufs1_v, bufr0_v, bufr1_v,
                 sg0, sg1, sw0, sw1):
    wid = lax.axis_index("s") * _NC + lax.axis_index("c")
    per_w = _E // _NW
    nchunks = per_w // _CG
    base = wid * per_w
    sg = (sg0, sg1)
    sw = (sw0, sw1)
    idxs = (idxs0_v, idxs1_v)
    idxr = (idxr0_v, idxr1_v)
    bufs = (bufs0_v, bufs1_v)
    bufr = (bufr0_v, bufr1_v)

    def launch(slot, off, drain_w):
        def _drain():
            pltpu.make_async_copy(bufs[slot], gs_hbm.at[pl.ds(base, _CG)],
                                  sw[slot]).wait()
            pltpu.make_async_copy(bufr[slot], gr_hbm.at[pl.ds(base, _CG)],
                                  sw[slot]).wait()

        _when(drain_w, _drain)
        pltpu.sync_copy(snd_hbm.at[pl.ds(off, _CG)], idxs[slot])
        pltpu.sync_copy(rcv_hbm.at[pl.ds(off, _CG)], idxr[slot])
        pltpu.async_copy(s_hbm.at[idxs[slot]], bufs[slot], sg[slot])
        pltpu.async_copy(r_hbm.at[idxr[slot]], bufr[slot], sg[slot])

    def finish(slot, off):
        pltpu.make_async_copy(s_hbm.at[idxs[slot]], bufs[slot], sg[slot]).wait()
        pltpu.make_async_copy(r_hbm.at[idxr[slot]], bufr[slot], sg[slot]).wait()
        pltpu.async_copy(bufs[slot], gs_hbm.at[pl.ds(off, _CG)], sw[slot])
        pltpu.async_copy(bufr[slot], gr_hbm.at[pl.ds(off, _CG)], sw[slot])

    launch(0, base, False)

    def pair(i, carry):
        off0 = base + (2 * i) * _CG
        launch(1, off0 + _CG, i > 0)
        finish(0, off0)
        launch(0, off0 + 2 * _CG, True)
        finish(1, off0 + _CG)
        return carry

    lax.fori_loop(0, (nchunks - 1) // 2, pair, 0)
    finish(0, base + (nchunks - 1) * _CG)
    for slot in (0, 1):
        pltpu.make_async_copy(bufs[slot], gs_hbm.at[pl.ds(base, _CG)],
                              sw[slot]).wait()
        pltpu.make_async_copy(bufr[slot], gr_hbm.at[pl.ds(base, _CG)],
                              sw[slot]).wait()


def _gather(S, R, senders, receivers):
    mesh = plsc.VectorSubcoreMesh(core_axis_name="c", subcore_axis_name="s",
                                  num_cores=_NC, num_subcores=_NS)
    kfn = pl.kernel(
        _gather_body,
        out_type=[jax.ShapeDtypeStruct((_E, _H2), jnp.int32),
                  jax.ShapeDtypeStruct((_E, _H2), jnp.int32)],
        mesh=mesh,
        scratch_types=[
            pltpu.VMEM((_CG,), jnp.int32),
            pltpu.VMEM((_CG,), jnp.int32),
            pltpu.VMEM((_CG,), jnp.int32),
            pltpu.VMEM((_CG,), jnp.int32),
            pltpu.VMEM((_CG, _H2), jnp.int32),
            pltpu.VMEM((_CG, _H2), jnp.int32),
            pltpu.VMEM((_CG, _H2), jnp.int32),
            pltpu.VMEM((_CG, _H2), jnp.int32),
            pltpu.SemaphoreType.DMA,
            pltpu.SemaphoreType.DMA,
            pltpu.SemaphoreType.DMA,
            pltpu.SemaphoreType.DMA,
        ],
    )
    return kfn(S, R, senders, receivers)


def _edge_body(gs_ref, gr_ref, ea_ref, wee_ref, bee_ref,
               we1al_ref, we1ah_ref, be1_ref,
               we2l_ref, we2h_ref, be2_ref, out_ref):
    bee = bee_ref[...]
    ea = ea_ref[...]
    slo, shi = _unpack_bf16_pair(gs_ref[...])
    rlo, rhi = _unpack_bf16_pair(gr_ref[...])

    weal = jnp.dot(wee_ref[...], we1al_ref[...], preferred_element_type=jnp.float32)
    weah = jnp.dot(wee_ref[...], we1ah_ref[...], preferred_element_type=jnp.float32)
    bl = jnp.dot(bee, we1al_ref[...], preferred_element_type=jnp.float32)
    bh = jnp.dot(bee, we1ah_ref[...], preferred_element_type=jnp.float32)
    bl = bl + be1_ref[:, :_H2]
    bh = bh + be1_ref[:, _H2:]

    hlo = slo + rlo + jnp.dot(ea, weal, preferred_element_type=jnp.float32) + bl
    hhi = shi + rhi + jnp.dot(ea, weah, preferred_element_type=jnp.float32) + bh
    hlo = jnp.maximum(hlo, 0.0)
    hhi = jnp.maximum(hhi, 0.0)
    out = jnp.dot(hlo, we2l_ref[...], preferred_element_type=jnp.float32)
    out = out + jnp.dot(hhi, we2h_ref[...], preferred_element_type=jnp.float32)
    out_ref[...] = out + be2_ref[...]


def _edge_mlp(Gs, Gr, edge_attr, W_ee, b_ee, We1a, be1, We2, be2):
    full = lambda shape: pl.BlockSpec(shape, lambda i: (0, 0))
    return pl.pallas_call(
        _edge_body,
        grid=(_EB,),
        in_specs=[
            pl.BlockSpec((_EBLK, _H2), lambda i: (i, 0)),
            pl.BlockSpec((_EBLK, _H2), lambda i: (i, 0)),
            pl.BlockSpec((_EBLK, 16), lambda i: (i, 0)),
            full((16, _LAT)),
            full((1, _LAT)),
            full((_LAT, _H2)),
            full((_LAT, _H2)),
            full((1, _H1)),
            full((_H2, _H2)),
            full((_H2, _H2)),
            full((1, _H2)),
        ],
        out_specs=pl.BlockSpec((_EBLK, _H2), lambda i: (i, 0)),
        out_shape=jax.ShapeDtypeStruct((_E, _H2), jnp.float32),
    )(Gs, Gr, edge_attr, W_ee, b_ee.reshape(1, _LAT),
      We1a[:, :_H2], We1a[:, _H2:], be1.reshape(1, _H1),
      We2[:_H2], We2[_H2:], be2.reshape(1, _H2))


_NPT = 632
_NPAD = _NS * _NPT


def _scatter_body(en_hbm, idx2_hbm, out_hbm, acc_sh,
                  ebuf0, ebuf1, idx0_v, idx1_v,
                  sl0, sl1, si0, si1, ss0, ss1):
    c = lax.axis_index("c")
    s = lax.axis_index("s")
    ebuf = (ebuf0, ebuf1)
    idxm = (idx0_v, idx1_v)
    slm = (sl0, sl1)
    sim = (si0, si1)
    ssm = (ss0, ss1)

    def zrow(rr, carry):
        for l in range(_H2 // 16):
            ebuf0[rr, pl.ds(l * 16, 16)] = jnp.zeros((16,), jnp.float32)
        return carry

    lax.fori_loop(0, _CS, zrow, 0)
    for k in range(_NPT // _CS):
        pltpu.sync_copy(ebuf0, acc_sh.at[pl.ds(s * _NPT + k * _CS, _CS)])
    rem = _NPT % _CS
    if rem:
        pltpu.sync_copy(ebuf0.at[pl.ds(0, rem)],
                        acc_sh.at[pl.ds(s * _NPT + (_NPT // _CS) * _CS, rem)])
    plsc.subcore_barrier()

    per_tile = _E // _NS
    nchunks = per_tile // _CS
    base = s * per_tile

    def launch(slot, off, drain):
        _when(drain, lambda: pltpu.make_async_copy(
            ebuf[slot], acc_sh.at[idxm[slot]], ssm[slot]).wait())
        pltpu.async_copy(idx2_hbm.at[pl.ds(c * _E + off, _CS)],
                         idxm[slot], sim[slot])
        pltpu.async_copy(en_hbm.at[pl.ds(off, _CS)], ebuf[slot], slm[slot])

    def finish(slot):
        pltpu.make_async_copy(idx2_hbm.at[pl.ds(c * _E + base, _CS)],
                              idxm[slot], sim[slot]).wait()
        pltpu.make_async_copy(en_hbm.at[pl.ds(base, _CS)], ebuf[slot],
                              slm[slot]).wait()
        pltpu.async_copy(ebuf[slot], acc_sh.at[idxm[slot]],
                         ssm[slot], add=True)

    launch(0, base, False)

    def pair(i, carry):
        off0 = base + (2 * i) * _CS
        launch(1, off0 + _CS, i > 0)
        finish(0)
        launch(0, off0 + 2 * _CS, True)
        finish(1)
        return carry

    lax.fori_loop(0, (nchunks - 1) // 2, pair, 0)
    finish(0)
    pltpu.make_async_copy(ebuf0, acc_sh.at[idx0_v], ssm[0]).wait()
    pltpu.make_async_copy(ebuf1, acc_sh.at[idx1_v], ssm[1]).wait()
    plsc.subcore_barrier()
    pltpu.sync_copy(acc_sh.at[pl.ds(s * _NPT, _NPT)],
                    out_hbm.at[c, pl.ds(s * _NPT, _NPT)])


def _scatter(edges_new, idx2):
    mesh = plsc.VectorSubcoreMesh(core_axis_name="c", subcore_axis_name="s",
                                  num_cores=_NC, num_subcores=_NS)
    kfn = pl.kernel(
        _scatter_body,
        out_type=jax.ShapeDtypeStruct((2, _NPAD, _H2), jnp.float32),
        mesh=mesh,
        scratch_types=[
            pltpu.VMEM_SHARED((_NPAD, _H2), jnp.float32),
            pltpu.VMEM((_CS, _H2), jnp.float32),
            pltpu.VMEM((_CS, _H2), jnp.float32),
            pltpu.VMEM((_CS,), jnp.int32),
            pltpu.VMEM((_CS,), jnp.int32),
            pltpu.SemaphoreType.DMA,
            pltpu.SemaphoreType.DMA,
            pltpu.SemaphoreType.DMA,
            pltpu.SemaphoreType.DMA,
            pltpu.SemaphoreType.DMA,
            pltpu.SemaphoreType.DMA,
        ],
    )
    return kfn(edges_new, idx2)


def _node_body(nodes_ref, sa_ref, ra_ref,
               wn1a_ref, wn1b_ref, wn1c_ref, bn1_ref,
               wn2_ref, bn2_ref, wg1a_ref, wg1b_ref, bg1_ref, wg2_ref, bg2_ref,
               wg3_ref, bg3_ref, out_ref, s1_acc, ea_acc):
    i = pl.program_id(0)

    @pl.when(i == 0)
    def _init():
        s1_acc[...] = jnp.zeros_like(s1_acc)
        ea_acc[...] = jnp.zeros_like(ea_acc)

    sa = sa_ref[...]
    h = jnp.dot(nodes_ref[...], wn1a_ref[...], preferred_element_type=jnp.float32)
    h = h + jnp.dot(sa, wn1b_ref[...], preferred_element_type=jnp.float32)
    h = h + jnp.dot(ra_ref[...], wn1c_ref[...], preferred_element_type=jnp.float32)
    h = jnp.maximum(h + bn1_ref[...], 0.0)
    s1_acc[...] = s1_acc[...] + jnp.sum(h, axis=0, keepdims=True)
    ea_acc[...] = ea_acc[...] + jnp.sum(sa, axis=0, keepdims=True)

    @pl.when(i == _NB - 1)
    def _final():
        node_agg = jnp.dot(s1_acc[...], wn2_ref[...],
                           preferred_element_type=jnp.float32)
        node_agg = node_agg + jnp.float32(_N) * bn2_ref[...]
        edge_agg = ea_acc[...]
        hg = jnp.dot(node_agg, wg1a_ref[...], preferred_element_type=jnp.float32)
        hg = hg + jnp.dot(edge_agg, wg1b_ref[...], preferred_element_type=jnp.float32)
        hg = jnp.maximum(hg + bg1_ref[...], 0.0)
        hg2 = jnp.dot(hg, wg2_ref[...], preferred_element_type=jnp.float32)
        hg2 = jnp.maximum(hg2 + bg2_ref[...], 0.0)
        out_ref[...] = (jnp.dot(hg2, wg3_ref[...], preferred_element_type=jnp.float32)
                        + bg3_ref[...])


def _node_global(nodes, sent_agg, recv_agg, Wn1a, Wn1b, Wn1c, bn1, Wn2, bn2,
                 Wg1a, Wg1b, bg1, Wg2, bg2, Wg3, bg3):
    full = lambda shape: pl.BlockSpec(shape, lambda i: (0, 0))
    return pl.pallas_call(
        _node_body,
        grid=(_NB,),
        in_specs=[
            pl.BlockSpec((_NBLK, _LAT), lambda i: (i, 0)),
            pl.BlockSpec((_NBLK, _H2), lambda i: (i, 0)),
            pl.BlockSpec((_NBLK, _H2), lambda i: (i, 0)),
            full((_LAT, _H1)),
            full((_H2, _H1)),
            full((_H2, _H1)),
            full((1, _H1)),
            full((_H1, _H2)),
            full((1, _H2)),
            full((_H2, _H1)),
            full((_H2, _H1)),
            full((1, _H1)),
            full((_H1, _H2)),
            full((1, _H2)),
            full((_H2, 1)),
            full((1, 1)),
        ],
        out_specs=pl.BlockSpec((1, 1), lambda i: (0, 0)),
        out_shape=jax.ShapeDtypeStruct((1, 1), jnp.float32),
        scratch_shapes=[
            pltpu.VMEM((1, _H1), jnp.float32),
            pltpu.VMEM((1, _H2), jnp.float32),
        ],
    )(nodes, sent_agg, recv_agg, Wn1a, Wn1b, Wn1c, bn1.reshape(1, _H1),
      Wn2, bn2.reshape(1, _H2), Wg1a, Wg1b, bg1.reshape(1, _H1),
      Wg2, bg2.reshape(1, _H2), Wg3, bg3.reshape(1, 1))


def kernel(x, edge_attr, senders, receivers, W_en, b_en, W_ee, b_ee,
           We1, be1, We2, be2, Wn1, bn1, Wn2, bn2,
           Wg1, bg1, Wg2, bg2, Wg3, bg3):
    We1a = We1[:_LAT]
    We1b = We1[_LAT:2 * _LAT]
    We1c = We1[2 * _LAT:3 * _LAT]
    Wn1a = Wn1[:_LAT]
    Wn1b = Wn1[_LAT:2 * _LAT]
    Wn1c = Wn1[2 * _LAT:3 * _LAT]
    Wg1a = Wg1[:_H2]
    Wg1b = Wg1[_H2:2 * _H2]

    nodes, S32, R32 = _prep(x, W_en, b_en, We1b, We1c)
    Gs32, Gr32 = _gather(S32, R32, senders, receivers)
    edges_new = _edge_mlp(Gs32, Gr32, edge_attr, W_ee, b_ee, We1a, be1, We2, be2)
    idx2 = jnp.concatenate([senders, receivers])
    aggs = _scatter(edges_new, idx2)
    out = _node_global(nodes, aggs[0, :_N], aggs[1, :_N], Wn1a, Wn1b, Wn1c, bn1, Wn2, bn2,
                       Wg1a, Wg1b, bg1, Wg2, bg2, Wg3, bg3)
    return out
